# Initial kernel scaffold; baseline (speedup 1.0000x reference)
import jax
import jax.numpy as jnp
from jax.experimental import pallas as pl


def kernel(features, conv_w, conv_b, obj_w, obj_b, delta_w, delta_b):
    raise NotImplementedError("write your pallas kernel here")


# speedup vs baseline: 7.8104x; 7.8104x over previous
import math

import jax
import jax.numpy as jnp
from jax.experimental import pallas as pl
from jax.experimental.pallas import tpu as pltpu

IMG = 1024.0
NMS_THRESH = 0.7
PRE_K = 2000
POST_K = 1000
SCALE_CLAMP = math.log(1000.0 / 16.0)

H = 128
W = 128
C = 256
WP = W + 2
NPIX = H * WP
NIN = (H + 2) * WP
NINP = NIN + 4
N = H * W * 3
CAP = 2048
BLK = 128
NBLK = CAP // BLK
NEG = -3.0e38
HIGH = jax.lax.Precision.HIGHEST

M_TILE = 1040
GRID_A = NPIX // M_TILE
TAPS = [(dy, dx) for dy in range(3) for dx in range(3)]


def _build_w9(conv_w):
    wt = jnp.transpose(conv_w, (2, 3, 1, 0))
    return jnp.concatenate([wt[dy, dx] for dy, dx in TAPS], axis=0)


def _fiota(shape, dim):
    return jax.lax.broadcasted_iota(jnp.int32, shape, dim).astype(jnp.float32)


def _conv_body(xp_hbm, w9_ref, b_ref, wh_ref, out_ref, xwin, sem):
    i = pl.program_id(0)
    m0 = i * M_TILE
    cp = pltpu.make_async_copy(xp_hbm.at[pl.ds(m0, M_TILE + 264), :], xwin, sem)
    cp.start()
    cp.wait()
    xcol = jnp.concatenate(
        [xwin[pl.ds(dy * WP + dx, M_TILE), :] for dy, dx in TAPS],
        axis=1)
    acc = jnp.dot(xcol, w9_ref[...], preferred_element_type=jnp.float32)
    t = jnp.maximum(acc + b_ref[0], 0.0)
    out_ref[...] = jnp.dot(t, wh_ref[...], preferred_element_type=jnp.float32)


def _conv_heads(xp_flat, w9, b, wh):
    return pl.pallas_call(
        _conv_body,
        grid=(GRID_A,),
        in_specs=[
            pl.BlockSpec(memory_space=pltpu.MemorySpace.HBM),
            pl.BlockSpec((9 * C, C), lambda i: (0, 0)),
            pl.BlockSpec((1, C), lambda i: (0, 0)),
            pl.BlockSpec((C, 16), lambda i: (0, 0)),
        ],
        out_specs=pl.BlockSpec((M_TILE, 16), lambda i: (i, 0)),
        out_shape=jax.ShapeDtypeStruct((NPIX, 16), jnp.float32),
        scratch_shapes=[pltpu.VMEM((M_TILE + 264, C), jnp.float32),
                        pltpu.SemaphoreType.DMA],
    )(xp_flat, w9, b, wh)


def _fkey(s):
    bits = jax.lax.bitcast_convert_type(s, jnp.int32)
    return bits ^ jnp.where(bits < 0, jnp.int32(0x7FFFFFFF), jnp.int32(0))


def _lane_prefix_ex(m):
    r = _fiota((128, 128), 0)
    c = _fiota((128, 128), 1)
    stl = (r < c).astype(jnp.float32)
    return jnp.dot(m, stl, preferred_element_type=jnp.float32)


def _row_offsets_ex(m):
    rs = jnp.sum(m, axis=1, keepdims=True)
    R = m.shape[0]
    r = _fiota((R, R), 0)
    c = _fiota((R, R), 1)
    stl = (c < r).astype(jnp.float32)
    return jnp.dot(stl, rs, preferred_element_type=jnp.float32)


def _prefix_ex(m):
    return _lane_prefix_ex(m) + _row_offsets_ex(m)


def _select_body(s_ref, d_ref, out_ref, pos_ref, sel_ref, acc_ref):
    keys = _fkey(s_ref[...])

    def cnt_ge(th):
        return jnp.sum((keys >= th).astype(jnp.int32))

    def bs_step(_, lohi):
        lo, hi = lohi
        mid = lo + ((hi >> 1) - (lo >> 1)) + (((hi & 1) - (lo & 1)) >> 1)
        big = cnt_ge(mid) >= PRE_K
        return jnp.where(big, mid, lo), jnp.where(big, hi, mid)

    lo0 = jnp.int32(-(2 ** 31))
    hi0 = jnp.int32(2 ** 31 - 1)
    tau, _ = jax.lax.fori_loop(0, 33, bs_step, (lo0, hi0))
    n_gt = jnp.sum((keys > tau).astype(jnp.int32))
    n_tie_take = (PRE_K - n_gt).astype(jnp.float32)

    hi_m = (keys > tau).astype(jnp.float32)
    tie_m = (keys == tau).astype(jnp.float32)
    tie_rank = _prefix_ex(tie_m)
    sel = hi_m + tie_m * (tie_rank < n_tie_take).astype(jnp.float32)
    pos = _prefix_ex(sel)

    slot_col = _fiota((CAP, 1), 0)
    pos_ref[...] = pos
    sel_ref[...] = sel
    acc_ref[...] = jnp.zeros((CAP, 8), jnp.float32)

    RCH = 8

    def chunk_step(cc, _):
        r0 = cc * RCH
        posb = pos_ref[pl.ds(r0, RCH), :]
        selb = sel_ref[pl.ds(r0, RCH), :]
        st = jnp.transpose(s_ref[pl.ds(r0, RCH), :])
        dts = [jnp.transpose(d_ref[pl.ds(k * 384 + r0, RCH), :])
               for k in range(4)]
        lane_col = _fiota((128, 1), 0)
        base = (r0 * 128).astype(jnp.float32)
        oh_parts = []
        val_parts = []
        for r in range(RCH):
            m_r = ((posb[r:r + 1, :] == slot_col)
                   & (selb[r:r + 1, :] > 0.5)).astype(jnp.float32)
            oh_parts.append(m_r)
            v_r = jnp.concatenate(
                [st[:, r:r + 1], lane_col + base + float(r * 128)]
                + [d[:, r:r + 1] for d in dts]
                + [jnp.zeros((128, 2), jnp.float32)], axis=1)
            val_parts.append(v_r)
        oh = jnp.concatenate(oh_parts, axis=1)
        vals = jnp.concatenate(val_parts, axis=0)
        acc_ref[...] = acc_ref[...] + jnp.dot(
            oh, vals, preferred_element_type=jnp.float32, precision=HIGH)
        return 0

    jax.lax.fori_loop(0, 384 // RCH, chunk_step, 0)
    acc = acc_ref[...]

    fake = slot_col >= float(PRE_K)
    score = jnp.where(fake, NEG, acc[:, 0:1])
    sidx = jnp.where(fake, 60000.0 + slot_col, acc[:, 1:2])

    sc_c = jnp.transpose(score)
    ix_c = jnp.transpose(sidx)
    SRC = 512
    ranks = []
    for q in range(CAP // SRC):
        s_q = score[q * SRC:(q + 1) * SRC]
        i_q = sidx[q * SRC:(q + 1) * SRC]
        beats = (sc_c > s_q) | ((sc_c == s_q) & (ix_c < i_q))
        ranks.append(jnp.sum(beats.astype(jnp.float32), axis=1, keepdims=True))
    rank_row = jnp.transpose(jnp.concatenate(ranks, axis=0))
    payload = jnp.concatenate([score, sidx, acc[:, 2:]], axis=1)
    for q in range(CAP // SRC):
        perm_q = (rank_row == (_fiota((SRC, CAP), 0)
                               + float(q * SRC))).astype(jnp.float32)
        out_ref[pl.ds(q * SRC, SRC), :] = jnp.dot(
            perm_q, payload, preferred_element_type=jnp.float32, precision=HIGH)


def _select_sort(scores, deltas):
    return pl.pallas_call(
        _select_body,
        in_specs=[pl.BlockSpec((N // 128, 128), lambda: (0, 0)),
                  pl.BlockSpec((N // 32, 128), lambda: (0, 0))],
        out_specs=pl.BlockSpec((CAP, 8), lambda: (0, 0)),
        out_shape=jax.ShapeDtypeStruct((CAP, 8), jnp.float32),
        scratch_shapes=[pltpu.VMEM((N // 128, 128), jnp.float32),
                        pltpu.VMEM((N // 128, 128), jnp.float32),
                        pltpu.VMEM((CAP, 8), jnp.float32)],
    )(scores.reshape(N // 128, 128), deltas)


def _nms_body(in_ref, out_ref):
    data = in_ref[...]
    score = data[:, 0:1]
    idx = data[:, 1:2].astype(jnp.int32)
    a = idx % 3
    pix = idx // 3
    gx = (pix % W).astype(jnp.float32)
    gy = (pix // W).astype(jnp.float32)
    cxa = gx * 8.0 + 4.0
    cya = gy * 8.0 + 4.0
    size = jnp.where(a == 0, 32.0, jnp.where(a == 1, 64.0, 128.0))
    dx = data[:, 2:3]
    dy = data[:, 3:4]
    dw = jnp.minimum(data[:, 4:5], SCALE_CLAMP)
    dh = jnp.minimum(data[:, 5:6], SCALE_CLAMP)
    cx = dx * size + cxa
    cy = dy * size + cya
    w = jnp.exp(dw) * size
    h = jnp.exp(dh) * size
    x1 = jnp.clip(cx - 0.5 * w, 0.0, IMG)
    y1 = jnp.clip(cy - 0.5 * h, 0.0, IMG)
    x2 = jnp.clip(cx + 0.5 * w, 0.0, IMG)
    y2 = jnp.clip(cy + 0.5 * h, 0.0, IMG)
    area = (x2 - x1) * (y2 - y1)

    x1r = jnp.transpose(x1)
    y1r = jnp.transpose(y1)
    x2r = jnp.transpose(x2)
    y2r = jnp.transpose(y2)
    arear = jnp.transpose(area)

    col = _fiota((1, CAP), 1)
    keep = (col < float(PRE_K)).astype(jnp.float32)

    rloc = _fiota((BLK, BLK), 0)
    cloc = _fiota((BLK, BLK), 1)
    upper = (rloc < cloc).astype(jnp.float32)

    for k in range(NBLK):
        b0 = k * BLK
        ltx = jnp.maximum(x1[b0:b0 + BLK], x1r)
        lty = jnp.maximum(y1[b0:b0 + BLK], y1r)
        rbx = jnp.minimum(x2[b0:b0 + BLK], x2r)
        rby = jnp.minimum(y2[b0:b0 + BLK], y2r)
        iw = jnp.maximum(rbx - ltx, 0.0)
        ih = jnp.maximum(rby - lty, 0.0)
        inter = iw * ih
        iou = inter / (area[b0:b0 + BLK] + arear - inter + 1e-9)
        adj = (iou > NMS_THRESH).astype(jnp.float32)
        adj_in = adj[:, b0:b0 + BLK] * upper
        m = keep[:, b0:b0 + BLK]

        def fix_step(state):
            kb, _ = state
            sup = jnp.dot(kb, adj_in, preferred_element_type=jnp.float32)
            kb2 = m * (sup == 0.0).astype(jnp.float32)
            return kb2, jnp.sum(jnp.abs(kb2 - kb))

        def fix_cond(state):
            return state[1] > 0.0

        kb, _ = jax.lax.while_loop(fix_cond, fix_step, (m, jnp.float32(1.0)))
        sup_all = jnp.dot(kb, adj, preferred_element_type=jnp.float32)
        parts = [] if b0 == 0 else [keep[:, :b0]]
        parts.append(kb)
        if b0 + BLK < CAP:
            parts.append(keep[:, b0 + BLK:]
                         * (1.0 - jnp.minimum(sup_all[:, b0 + BLK:], 1.0)))
        keep = jnp.concatenate(parts, axis=1) if len(parts) > 1 else parts[0]

    def scan_ex(x):
        s = jnp.concatenate([jnp.zeros((1, 1), jnp.float32), x[:, :-1]], axis=1)
        sh = 1
        while sh < CAP:
            s = s + jnp.concatenate(
                [jnp.zeros((1, sh), jnp.float32), s[:, :-sh]], axis=1)
            sh *= 2
        return s

    pos_k = scan_ex(keep)
    pos_s = scan_ex(1.0 - keep)
    nkept = jnp.sum(keep)
    pos2 = jnp.where(keep > 0.5, pos_k, nkept + pos_s)
    pos2 = jnp.where(col < float(PRE_K), pos2, 3.0e6)

    vals = jnp.concatenate([x1, y1, x2, y2, score,
                            jnp.zeros((CAP, 3), jnp.float32)], axis=1)
    for q in range(2):
        oh_q = (pos2 == (_fiota((512, CAP), 0) + float(q * 512))).astype(jnp.float32)
        out_ref[pl.ds(q * 512, 512), :] = jnp.dot(
            oh_q, vals, preferred_element_type=jnp.float32, precision=HIGH)


def _nms_final(sorted_payload):
    return pl.pallas_call(
        _nms_body,
        in_specs=[pl.BlockSpec((CAP, 8), lambda: (0, 0))],
        out_specs=pl.BlockSpec((1024, 8), lambda: (0, 0)),
        out_shape=jax.ShapeDtypeStruct((1024, 8), jnp.float32),
    )(sorted_payload)


def kernel(features, conv_w, conv_b, obj_w, obj_b, delta_w, delta_b):
    x = jnp.transpose(features[0], (1, 2, 0))
    xp = jnp.pad(x, ((1, 1), (1, 1), (0, 0)))
    xp_flat = jnp.pad(xp.reshape(NIN, C), ((0, NINP - NIN), (0, 0)))
    w9 = _build_w9(conv_w)
    wh = jnp.concatenate([
        jnp.transpose(obj_w[:, :, 0, 0]),
        jnp.transpose(delta_w[:, :, 0, 0]),
        jnp.zeros((C, 1), jnp.float32),
    ], axis=1)
    bh = jnp.concatenate([obj_b, delta_b, jnp.zeros((1,), jnp.float32)])

    heads = _conv_heads(xp_flat, w9, conv_b.reshape(1, C), wh)
    heads = heads.reshape(H, WP, 16)[:, :W, :] + bh[None, None, :]
    scores = heads[:, :, 0:3].reshape(N)
    deltas = heads[:, :, 3:15].reshape(N, 4)
    dplanes = jnp.transpose(deltas).reshape(4 * (N // 128), 128)

    sorted_payload = _select_sort(scores, dplanes)
    out = _nms_final(sorted_payload)
    return out[:POST_K, :5]
